# baseline (speedup 1.0000x reference)
import functools

import jax
import jax.numpy as jnp
from jax import lax
from jax.experimental import pallas as pl
from jax.experimental.pallas import tpu as pltpu
from jax.experimental.pallas import tpu_sc as plsc

D = 128
SEQ = 200
BATCH = 1024
NC = 2
NS = 16
NW = NC * NS
ROWS_W = BATCH * SEQ // NW
CHUNK = 128
NCH = ROWS_W // CHUNK
LANES = 16
NBUF = 6
LEAD = 4
LAG = 2


def _body(idx_hbm, table_hbm, pe_hbm, out_hbm,
          idx_v, pe_v, buf0, buf1, buf2, buf3, buf4, buf5,
          sem0, sem1, sem2, sem3, sem4, sem5):
    cid = lax.axis_index("c")
    sid = lax.axis_index("s")
    wid = sid * NC + cid

    pltpu.sync_copy(idx_hbm.at[wid], idx_v)
    pltpu.sync_copy(pe_hbm.at[pl.ds(0, SEQ)], pe_v)

    bufs = (buf0, buf1, buf2, buf3, buf4, buf5)
    sems = (sem0, sem1, sem2, sem3, sem4, sem5)

    def start_gather(c):
        b = c % NBUF
        return pltpu.async_copy(table_hbm.at[idx_v.at[c]], bufs[b], sems[b])

    gathers = {c: start_gather(c) for c in range(LEAD)}
    writes = {}

    for c in range(NCH):
        b = c % NBUF
        buf = bufs[b]
        gathers.pop(c).wait()

        p0 = (c * CHUNK) % SEQ
        n_first = min(SEQ - p0, CHUNK)

        def seg(lo, hi, pe_off):
            def add_pe(r, carry):
                for cc in range(D // LANES):
                    sl = pl.ds(cc * LANES, LANES)
                    plsc.addupdate(buf.at[r, sl], pe_v[r + pe_off, sl])
                return carry
            lax.fori_loop(lo, hi, add_pe, 0)

        seg(0, n_first, p0)
        if n_first < CHUNK:
            seg(n_first, CHUNK, -n_first)

        writes[c] = pltpu.async_copy(buf, out_hbm.at[wid * NCH + c], sems[b])

        if c + LEAD < NCH:
            if c - LAG >= 0:
                writes.pop(c - LAG).wait()
            gathers[c + LEAD] = start_gather(c + LEAD)

    for c in sorted(writes):
        writes.pop(c).wait()


_emb = functools.partial(
    pl.kernel,
    out_type=jax.ShapeDtypeStruct((NW * NCH, CHUNK, D), jnp.float32),
    mesh=plsc.VectorSubcoreMesh(core_axis_name="c", subcore_axis_name="s"),
    scratch_types=[
        pltpu.VMEM((NCH, CHUNK), jnp.int32),
        pltpu.VMEM((SEQ, D), jnp.float32),
        pltpu.VMEM((CHUNK, D), jnp.float32),
        pltpu.VMEM((CHUNK, D), jnp.float32),
        pltpu.VMEM((CHUNK, D), jnp.float32),
        pltpu.VMEM((CHUNK, D), jnp.float32),
        pltpu.VMEM((CHUNK, D), jnp.float32),
        pltpu.VMEM((CHUNK, D), jnp.float32),
        pltpu.SemaphoreType.DMA,
        pltpu.SemaphoreType.DMA,
        pltpu.SemaphoreType.DMA,
        pltpu.SemaphoreType.DMA,
        pltpu.SemaphoreType.DMA,
        pltpu.SemaphoreType.DMA,
    ],
)(_body)


@jax.jit
def kernel(inputs, table, pos_encoding):
    idx = inputs.astype(jnp.int32).reshape(NW, NCH, CHUNK)
    out = _emb(idx, table, pos_encoding)
    return out.reshape(BATCH, SEQ, D)


# speedup vs baseline: 1.0150x; 1.0150x over previous
import functools

import jax
import jax.numpy as jnp
from jax import lax
from jax.experimental import pallas as pl
from jax.experimental.pallas import tpu as pltpu
from jax.experimental.pallas import tpu_sc as plsc

D = 128
SEQ = 200
BATCH = 1024
NC = 2
NS = 16
NW = NC * NS
ROWS_W = BATCH * SEQ // NW
SPLIT = 128
CHUNK = 2 * SPLIT
NCH = ROWS_W // CHUNK
LANES = 16
NBUF = 3


def _body(idx_hbm, table_hbm, pe_hbm, out_hbm,
          idx_v, pe_v, buf0, buf1, buf2, sem0, sem1, sem2):
    cid = lax.axis_index("c")
    sid = lax.axis_index("s")
    wid = sid * NC + cid

    pltpu.sync_copy(idx_hbm.at[wid], idx_v)
    pltpu.sync_copy(pe_hbm.at[pl.ds(0, SEQ)], pe_v)

    bufs = (buf0, buf1, buf2)
    sems = (sem0, sem1, sem2)

    def start_gather(c):
        b = c % NBUF
        c0 = pltpu.async_copy(
            table_hbm.at[idx_v.at[2 * c]],
            bufs[b].at[pl.ds(0, SPLIT)], sems[b])
        c1 = pltpu.async_copy(
            table_hbm.at[idx_v.at[2 * c + 1]],
            bufs[b].at[pl.ds(SPLIT, SPLIT)], sems[b])
        return c0, c1

    gathers = {0: start_gather(0), 1: start_gather(1)}
    writes = {}

    for c in range(NCH):
        b = c % NBUF
        buf = bufs[b]
        for d in gathers.pop(c):
            d.wait()

        def seg(lo, hi, pe_off):
            def add_pe(r, carry):
                for cc in range(D // LANES):
                    sl = pl.ds(cc * LANES, LANES)
                    plsc.addupdate(buf.at[r, sl], pe_v[r + pe_off, sl])
                return carry
            lax.fori_loop(lo, hi, add_pe, 0)

        r0 = 0
        pos = (c * CHUNK) % SEQ
        while r0 < CHUNK:
            n = min(SEQ - pos, CHUNK - r0)
            seg(r0, r0 + n, pos - r0)
            r0 += n
            pos = (pos + n) % SEQ

        writes[c] = pltpu.async_copy(buf, out_hbm.at[wid * NCH + c], sems[b])

        if c + 2 < NCH:
            if c - 1 >= 0:
                writes.pop(c - 1).wait()
            gathers[c + 2] = start_gather(c + 2)

    for c in sorted(writes):
        writes.pop(c).wait()


_emb = functools.partial(
    pl.kernel,
    out_type=jax.ShapeDtypeStruct((NW * NCH, CHUNK, D), jnp.float32),
    mesh=plsc.VectorSubcoreMesh(core_axis_name="c", subcore_axis_name="s"),
    scratch_types=[
        pltpu.VMEM((2 * NCH, SPLIT), jnp.int32),
        pltpu.VMEM((SEQ, D), jnp.float32),
        pltpu.VMEM((CHUNK, D), jnp.float32),
        pltpu.VMEM((CHUNK, D), jnp.float32),
        pltpu.VMEM((CHUNK, D), jnp.float32),
        pltpu.SemaphoreType.DMA,
        pltpu.SemaphoreType.DMA,
        pltpu.SemaphoreType.DMA,
    ],
)(_body)


@jax.jit
def kernel(inputs, table, pos_encoding):
    idx = inputs.astype(jnp.int32).reshape(NW, 2 * NCH, SPLIT)
    out = _emb(idx, table, pos_encoding)
    return out.reshape(BATCH, SEQ, D)


# speedup vs baseline: 1.0206x; 1.0055x over previous
import functools

import jax
import jax.numpy as jnp
from jax import lax
from jax.experimental import pallas as pl
from jax.experimental.pallas import tpu as pltpu
from jax.experimental.pallas import tpu_sc as plsc

D = 128
SEQ = 200
BATCH = 1024
NC = 2
NS = 16
NW = NC * NS
ROWS_W = BATCH * SEQ // NW
SPLIT = 128
CHUNK = 2 * SPLIT
NCH = ROWS_W // CHUNK
LANES = 16
NBUF = 3


def _body(idx_hbm, table_hbm, pe_hbm, out_hbm,
          idx_v, pe_v, buf0, buf1, buf2, sem0, sem1, sem2, sem3):
    cid = lax.axis_index("c")
    sid = lax.axis_index("s")
    wid = sid * NC + cid

    pltpu.sync_copy(idx_hbm.at[wid], idx_v)

    bufs = (buf0, buf1, buf2)
    sems = (sem0, sem1, sem2)

    def start_gather(c):
        b = c % NBUF
        c0 = pltpu.async_copy(
            table_hbm.at[idx_v.at[2 * c]],
            bufs[b].at[pl.ds(0, SPLIT)], sems[b])
        c1 = pltpu.async_copy(
            table_hbm.at[idx_v.at[2 * c + 1]],
            bufs[b].at[pl.ds(SPLIT, SPLIT)], sems[b])
        return c0, c1

    gathers = {0: start_gather(0), 1: start_gather(1)}
    writes = {}

    pe_copy = pltpu.async_copy(pe_hbm.at[pl.ds(0, SEQ)], pe_v, sem3)

    for c in range(NCH):
        b = c % NBUF
        buf = bufs[b]
        for d in gathers.pop(c):
            d.wait()
        if c == 0:
            pe_copy.wait()

        def seg(lo, hi, pe_off):
            def add_pe(r, carry):
                for cc in range(D // LANES):
                    sl = pl.ds(cc * LANES, LANES)
                    plsc.addupdate(buf.at[r, sl], pe_v[r + pe_off, sl])
                return carry
            lax.fori_loop(lo, hi, add_pe, 0)

        r0 = 0
        pos = (c * CHUNK) % SEQ
        while r0 < CHUNK:
            n = min(SEQ - pos, CHUNK - r0)
            seg(r0, r0 + n, pos - r0)
            r0 += n
            pos = (pos + n) % SEQ

        writes[c] = pltpu.async_copy(buf, out_hbm.at[wid * NCH + c], sems[b])

        if c + 2 < NCH:
            if c - 1 >= 0:
                writes.pop(c - 1).wait()
            gathers[c + 2] = start_gather(c + 2)

    for c in sorted(writes):
        writes.pop(c).wait()


_emb = functools.partial(
    pl.kernel,
    out_type=jax.ShapeDtypeStruct((NW * NCH, CHUNK, D), jnp.float32),
    mesh=plsc.VectorSubcoreMesh(core_axis_name="c", subcore_axis_name="s"),
    scratch_types=[
        pltpu.VMEM((2 * NCH, SPLIT), jnp.int32),
        pltpu.VMEM((SEQ, D), jnp.float32),
        pltpu.VMEM((CHUNK, D), jnp.float32),
        pltpu.VMEM((CHUNK, D), jnp.float32),
        pltpu.VMEM((CHUNK, D), jnp.float32),
        pltpu.SemaphoreType.DMA,
        pltpu.SemaphoreType.DMA,
        pltpu.SemaphoreType.DMA,
        pltpu.SemaphoreType.DMA,
    ],
)(_body)


@jax.jit
def kernel(inputs, table, pos_encoding):
    idx = inputs.astype(jnp.int32).reshape(NW, 2 * NCH, SPLIT)
    out = _emb(idx, table, pos_encoding)
    return out.reshape(BATCH, SEQ, D)
